# baseline (speedup 1.0000x reference)
import functools

import numpy as np
import jax
import jax.numpy as jnp
from jax import lax
from jax.experimental import pallas as pl
from jax.experimental.pallas import tpu as pltpu
from jax.experimental.pallas import tpu_sc as plsc

KAPPA = 0.9
NC, NS, LANES = 2, 16, 16
NW = NC * NS
CE = 128
CH = 80
NITER = 50
NP = 10240
BLK = 512

_GDN = lax.GatherDimensionNumbers(
    offset_dims=(), collapsed_slice_dims=(0,), start_index_map=(0,))


def _splat(vec, lane):
    idx = jnp.full((LANES, 1), lane, jnp.int32)
    return lax.gather(vec, idx, _GDN, slice_sizes=(1,),
                      mode=lax.GatherScatterMode.PROMISE_IN_BOUNDS)


def _rsq(x):
    xs = jnp.maximum(x, np.float32(1e-30))
    i = lax.bitcast_convert_type(xs, jnp.int32)
    i = np.int32(0x5F3759DF) - lax.shift_right_logical(i, 1)
    y = lax.bitcast_convert_type(i, jnp.float32)
    for _ in range(4):
        y = y * (np.float32(1.5) - np.float32(0.5) * xs * y * y)
    return y


@functools.cache
def _make_spmm(n, m, split):
    mesh = plsc.VectorSubcoreMesh(core_axis_name="c", subcore_axis_name="s")
    mh = m // 2 if split else m
    cht = 2 * CH if split else CH
    rpt = n // NS
    grp = mh // LANES

    def body(y_hbm, row_hbm, col_hbm, w_hbm, out_hbm,
             rowt, colt, wt, ga, gb, gsa, gsb, ssa, ssb, accum):
        c = lax.axis_index("c")
        s = lax.axis_index("s")
        if split:
            pltpu.sync_copy(row_hbm.at[s], rowt)
            pltpu.sync_copy(col_hbm.at[s], colt)
            pltpu.sync_copy(w_hbm.at[s], wt)
            ysrc = y_hbm.at[c]
        else:
            pltpu.sync_copy(row_hbm.at[s, pl.ds(c * CH, CH)], rowt)
            pltpu.sync_copy(col_hbm.at[s, pl.ds(c * CH, CH)], colt)
            pltpu.sync_copy(w_hbm.at[s, pl.ds(c * CH, CH)], wt)
            ysrc = y_hbm

        def _z(i, _):
            for j in range(grp):
                ga[i, pl.ds(j * LANES, LANES)] = jnp.zeros((LANES,), jnp.float32)
            return 0
        lax.fori_loop(0, CE, _z, 0)
        base = s * rpt
        for q in range(rpt // CE):
            pltpu.sync_copy(ga, accum.at[pl.ds(base + q * CE, CE)])
        pltpu.async_copy(ysrc.at[rowt.at[0]], ga, gsa)
        pltpu.async_copy(ysrc.at[rowt.at[1]], gb, gsb)
        plsc.subcore_barrier()

        def scale(buf, k):
            def grp16(g, _):
                wv = wt[k, pl.ds(g * LANES, LANES)]
                for lane in range(LANES):
                    ws = _splat(wv, lane)
                    e = g * LANES + lane
                    for j in range(grp):
                        buf[e, pl.ds(j * LANES, LANES)] = (
                            buf[e, pl.ds(j * LANES, LANES)] * ws)
                return 0
            lax.fori_loop(0, CE // LANES, grp16, 0)

        def pair(t, _):
            ka = 2 * t
            kb = 2 * t + 1
            pltpu.make_async_copy(ysrc.at[rowt.at[ka]], ga, gsa).wait()
            scale(ga, ka)
            pltpu.async_copy(ga, accum.at[colt.at[ka]], ssa, add=True)
            pltpu.make_async_copy(ysrc.at[rowt.at[kb]], gb, gsb).wait()
            scale(gb, kb)
            pltpu.async_copy(gb, accum.at[colt.at[kb]], ssb, add=True)
            ka2 = jnp.minimum(ka + 2, cht - 2)
            kb2 = jnp.minimum(kb + 2, cht - 1)
            pltpu.make_async_copy(ga, accum.at[colt.at[ka]], ssa).wait()
            pltpu.async_copy(ysrc.at[rowt.at[ka2]], ga, gsa)
            pltpu.make_async_copy(gb, accum.at[colt.at[kb]], ssb).wait()
            pltpu.async_copy(ysrc.at[rowt.at[kb2]], gb, gsb)
            return 0
        lax.fori_loop(0, cht // 2, pair, 0)
        pltpu.make_async_copy(ysrc.at[rowt.at[cht - 2]], ga, gsa).wait()
        pltpu.make_async_copy(ysrc.at[rowt.at[cht - 1]], gb, gsb).wait()
        plsc.subcore_barrier()
        pltpu.sync_copy(accum.at[pl.ds(base, rpt)],
                        out_hbm.at[c, pl.ds(base, rpt)])

    return pl.kernel(
        body,
        out_type=jax.ShapeDtypeStruct((NC, n, mh), jnp.float32),
        mesh=mesh,
        compiler_params=pltpu.CompilerParams(
            needs_layout_passes=False, use_tc_tiling_on_sc=False),
        scratch_types=[
            pltpu.VMEM((cht, CE), jnp.int32),
            pltpu.VMEM((cht, CE), jnp.int32),
            pltpu.VMEM((cht, CE), jnp.float32),
            pltpu.VMEM((CE, mh), jnp.float32),
            pltpu.VMEM((CE, mh), jnp.float32),
            pltpu.SemaphoreType.DMA,
            pltpu.SemaphoreType.DMA,
            pltpu.SemaphoreType.DMA,
            pltpu.SemaphoreType.DMA,
            pltpu.VMEM_SHARED((n, mh), jnp.float32),
        ],
    )


@functools.cache
def _make_rho(n):
    mesh = plsc.VectorSubcoreMesh(core_axis_name="c", subcore_axis_name="s")
    npad = NP
    spt = npad // NS

    def body(row_hbm, col_hbm, w_hbm, out_hbm,
             rowt, colt, wt, pb, vt, avt, zb, sb, pbt, sem, ssem, accum, parts):
        c = lax.axis_index("c")
        s = lax.axis_index("s")
        pltpu.sync_copy(row_hbm.at[s], rowt)
        pltpu.sync_copy(col_hbm.at[s], colt)
        pltpu.sync_copy(w_hbm.at[s], wt)

        v0 = jnp.full((LANES,), np.float32(1.0) / np.float32(np.sqrt(n)),
                      jnp.float32)
        zv = jnp.zeros((LANES,), jnp.float32)

        def _iv(i, _):
            vt[pl.ds(i * LANES, LANES)] = v0
            return 0
        lax.fori_loop(0, n // LANES, _iv, 0)

        def _ivz(i, _):
            vt[pl.ds(i * LANES, LANES)] = zv
            return 0
        lax.fori_loop(n // LANES, npad // LANES, _ivz, 0)
        for i in range(spt // LANES):
            zb[pl.ds(i * LANES, LANES)] = zv

        def it(t, s2_prev):
            pltpu.sync_copy(zb, accum.at[pl.ds(s * spt, spt)])
            plsc.subcore_barrier()

            def prow(r, _):
                for j in range(CE // LANES):
                    cv = colt[r, pl.ds(j * LANES, LANES)]
                    wv = wt[r, pl.ds(j * LANES, LANES)]
                    vv = plsc.load_gather(vt, [cv])
                    pb[r, pl.ds(j * LANES, LANES)] = vv * wv
                pltpu.async_copy(pb.at[r], accum.at[rowt.at[r]], ssem,
                                 add=True)
                return 0
            lax.fori_loop(0, 2 * CH, prow, 0)
            for r in range(2 * CH):
                pltpu.make_async_copy(pb.at[r], accum.at[rowt.at[r]],
                                      ssem).wait()
            plsc.subcore_barrier()

            pltpu.sync_copy(accum, avt)

            def sq(i, acc):
                x = avt[pl.ds(s * spt + i * LANES, LANES)]
                return acc + x * x
            part = lax.fori_loop(0, spt // LANES, sq,
                                 jnp.zeros((LANES,), jnp.float32))
            sb[...] = part
            pltpu.sync_copy(sb, parts.at[s])
            plsc.subcore_barrier()
            pltpu.sync_copy(parts, pbt)
            tot = jnp.zeros((LANES,), jnp.float32)
            for t2 in range(NS):
                tot = tot + pbt[t2]
            s2 = jnp.sum(tot)
            inv = _rsq(s2)

            def up(i, _):
                vt[pl.ds(i * LANES, LANES)] = avt[pl.ds(i * LANES, LANES)] * inv
                return 0
            lax.fori_loop(0, npad // LANES, up, 0)
            plsc.subcore_barrier()
            return s2
        s2f = lax.fori_loop(0, NITER, it, jnp.float32(0.0))
        rho = s2f * _rsq(s2f)

        @pl.when((c == 0) & (s == 0))
        def _():
            sb[...] = jnp.full((LANES,), np.float32(0.0)) + rho
            pltpu.sync_copy(sb, out_hbm)

    return pl.kernel(
        body,
        out_type=jax.ShapeDtypeStruct((LANES,), jnp.float32),
        mesh=mesh,
        compiler_params=pltpu.CompilerParams(
            needs_layout_passes=False, use_tc_tiling_on_sc=False),
        scratch_types=[
            pltpu.VMEM((2 * CH, CE), jnp.int32),
            pltpu.VMEM((2 * CH, CE), jnp.int32),
            pltpu.VMEM((2 * CH, CE), jnp.float32),
            pltpu.VMEM((2 * CH, CE), jnp.float32),
            pltpu.VMEM((npad,), jnp.float32),
            pltpu.VMEM((npad,), jnp.float32),
            pltpu.VMEM((spt,), jnp.float32),
            pltpu.VMEM((LANES,), jnp.float32),
            pltpu.VMEM((NS, LANES), jnp.float32),
            pltpu.SemaphoreType.DMA,
            pltpu.SemaphoreType.DMA,
            pltpu.VMEM_SHARED((npad,), jnp.float32),
            pltpu.VMEM_SHARED((NS, LANES), jnp.float32),
        ],
    )


def _store(o_ref, res, split, mh):
    if split:
        o_ref[0] = res[:, :mh]
        o_ref[1] = res[:, mh:]
    else:
        o_ref[...] = res


def _merge2(a_ref, b_ref, split):
    if split:
        return jnp.concatenate([a_ref[0] + b_ref[0], a_ref[1] + b_ref[1]],
                               axis=-1)
    return a_ref[0] + a_ref[1] + b_ref[0] + b_ref[1]


def _merge1(a_ref, split):
    if split:
        return jnp.concatenate([a_ref[0], a_ref[1]], axis=-1)
    return a_ref[0] + a_ref[1]


@functools.cache
def _mm(n, kin, kout, split):
    mh = kout // 2 if split else kout

    def body(z_ref, w_ref, o_ref):
        res = jnp.dot(z_ref[...], w_ref[...],
                      preferred_element_type=jnp.float32)
        _store(o_ref, res, split, mh)
    if split:
        out_spec = pl.BlockSpec((NC, BLK, mh), lambda i: (0, i, 0))
        out_shape = jax.ShapeDtypeStruct((NC, n, mh), jnp.float32)
    else:
        out_spec = pl.BlockSpec((BLK, kout), lambda i: (i, 0))
        out_shape = jax.ShapeDtypeStruct((n, kout), jnp.float32)
    return pl.pallas_call(
        body,
        grid=(n // BLK,),
        in_specs=[pl.BlockSpec((BLK, kin), lambda i: (i, 0)),
                  pl.BlockSpec((kin, kout), lambda i: (0, 0))],
        out_specs=out_spec,
        out_shape=out_shape,
    )


@functools.cache
def _relu_mm(n, m, with_p, split):
    mh = m // 2 if split else m

    def body(*refs):
        if with_p:
            p_ref, b_ref, w_ref, o_ref = refs
            x = _merge2(p_ref, b_ref, split)
        else:
            b_ref, w_ref, o_ref = refs
            x = _merge1(b_ref, split)
        res = jnp.dot(jnp.maximum(x, 0.0), w_ref[...],
                      preferred_element_type=jnp.float32)
        _store(o_ref, res, split, mh)
    pspec = pl.BlockSpec((NC, BLK, mh), lambda i: (0, i, 0))
    in_specs = ([pspec, pspec] if with_p else [pspec]) + [
        pl.BlockSpec((m, m), lambda i: (0, 0))]
    if split:
        out_spec = pl.BlockSpec((NC, BLK, mh), lambda i: (0, i, 0))
        out_shape = jax.ShapeDtypeStruct((NC, n, mh), jnp.float32)
    else:
        out_spec = pl.BlockSpec((BLK, m), lambda i: (i, 0))
        out_shape = jax.ShapeDtypeStruct((n, m), jnp.float32)
    return pl.pallas_call(
        body,
        grid=(n // BLK,),
        in_specs=in_specs,
        out_specs=out_spec,
        out_shape=out_shape,
    )


@functools.cache
def _comb(n, m, kin, act, split):
    mh = m // 2 if split else m

    def body(p_ref, b_ref, z_ref, w_ref, bias_ref, o_ref):
        x = jnp.maximum(_merge2(p_ref, b_ref, split), 0.0)
        y = x + jnp.dot(z_ref[...], w_ref[...],
                        preferred_element_type=jnp.float32) + bias_ref[...]
        o_ref[...] = jnp.where(y > 0, y, jnp.exp(y) - 1.0) if act else y
    pspec = pl.BlockSpec((NC, BLK, mh), lambda i: (0, i, 0))
    return pl.pallas_call(
        body,
        grid=(n // BLK,),
        in_specs=[pspec, pspec,
                  pl.BlockSpec((BLK, kin), lambda i: (i, 0)),
                  pl.BlockSpec((kin, m), lambda i: (0, 0)),
                  pl.BlockSpec((1, m), lambda i: (0, 0))],
        out_specs=pl.BlockSpec((BLK, m), lambda i: (i, 0)),
        out_shape=jax.ShapeDtypeStruct((n, m), jnp.float32),
    )


def _proj(W, v):
    a = jnp.abs(W)
    asort = jnp.sort(a, axis=1)[:, ::-1]
    cssv = jnp.cumsum(asort, axis=1) - v
    ind = jnp.arange(1, W.shape[1] + 1, dtype=W.dtype)
    cond = (asort - cssv / ind) > 0
    rho = jnp.maximum(jnp.sum(cond, axis=1), 1)
    theta = cssv[jnp.arange(W.shape[0]), rho - 1] / rho.astype(W.dtype)
    proj = jnp.sign(W) * jnp.maximum(a - theta[:, None], 0.0)
    return jnp.where((jnp.sum(a, axis=1) > v)[:, None], proj, W)


def _pack_edges(x, pad):
    x = jnp.pad(x.reshape(NW, -1), ((0, 0), (0, pad))).reshape(NW, CH, CE)
    return x.reshape(2, NS, CH, CE).transpose(1, 0, 2, 3).reshape(NS, 2 * CH, CE)


def kernel(features, edge_index, edge_weight, W1, O1, W2, O2, W3, O3, W4, O4,
           W5, O5, V0w, V0b, V1w, V1b, V2w, V2b, V3w, V3b, Vw, Vb):
    n = features.shape[1]
    e = edge_weight.shape[0]
    pad = CH * CE - e // NW
    rowg = _pack_edges(edge_index[0], pad)
    colg = _pack_edges(edge_index[1], pad)
    wg = _pack_edges(edge_weight, pad)

    a_rho = _make_rho(n)(rowg, colg, wg)[0]
    radius = KAPPA / a_rho

    z = jnp.pad(features.T, ((0, NP - n), (0, 0)))
    layers = [(W1, O1, V0w, V0b), (W2, O2, V1w, V1b), (W3, O3, V2w, V2b),
              (W4, O4, V3w, V3b), (W5, O5, Vw, Vb)]
    for li, (W, O, Vw_, Vb_) in enumerate(layers):
        m, p = O.shape
        split = m >= 32
        Wp = _proj(W, radius)
        spmm = _make_spmm(NP, m, split)
        s2 = _mm(NP, p, m, split)(z, O.T)
        b2 = spmm(s2, rowg, colg, wg)
        h = _relu_mm(NP, m, False, split)(b2, Wp.T)
        for _ in range(8):
            p2 = spmm(h, rowg, colg, wg)
            h = _relu_mm(NP, m, True, split)(p2, b2, Wp.T)
        p2 = spmm(h, rowg, colg, wg)
        z = _comb(NP, m, p, li < 4, split)(p2, b2, z, Vw_.T, Vb_.reshape(1, m))
    return z[:n]


# speedup vs baseline: 1.1520x; 1.1520x over previous
import functools

import numpy as np
import jax
import jax.numpy as jnp
from jax import lax
from jax.experimental import pallas as pl
from jax.experimental.pallas import tpu as pltpu
from jax.experimental.pallas import tpu_sc as plsc

KAPPA = 0.9
NC, NS, LANES = 2, 16, 16
NW = NC * NS
CE = 128
CH = 80
NITER = 50
NP = 10240
BLK = 512

_GDN = lax.GatherDimensionNumbers(
    offset_dims=(), collapsed_slice_dims=(0,), start_index_map=(0,))


def _splat(vec, lane):
    idx = jnp.full((LANES, 1), lane, jnp.int32)
    return lax.gather(vec, idx, _GDN, slice_sizes=(1,),
                      mode=lax.GatherScatterMode.PROMISE_IN_BOUNDS)


def _rsq(x):
    xs = jnp.maximum(x, np.float32(1e-30))
    i = lax.bitcast_convert_type(xs, jnp.int32)
    i = np.int32(0x5F3759DF) - lax.shift_right_logical(i, 1)
    y = lax.bitcast_convert_type(i, jnp.float32)
    for _ in range(4):
        y = y * (np.float32(1.5) - np.float32(0.5) * xs * y * y)
    return y


@functools.cache
def _make_spmm(n, m, split):
    mesh = plsc.VectorSubcoreMesh(core_axis_name="c", subcore_axis_name="s")
    mh = m // 2 if split else m
    cht = 2 * CH if split else CH
    rpt = n // NS
    grp = mh // LANES

    def body(y_hbm, row_hbm, col_hbm, w_hbm, out_hbm,
             rowt, colt, wt, ga, gb, gsa, gsb, ssa, ssb, accum):
        c = lax.axis_index("c")
        s = lax.axis_index("s")
        if split:
            pltpu.sync_copy(row_hbm.at[s], rowt)
            pltpu.sync_copy(col_hbm.at[s], colt)
            pltpu.sync_copy(w_hbm.at[s], wt)
            ysrc = y_hbm.at[c]
        else:
            pltpu.sync_copy(row_hbm.at[s, pl.ds(c * CH, CH)], rowt)
            pltpu.sync_copy(col_hbm.at[s, pl.ds(c * CH, CH)], colt)
            pltpu.sync_copy(w_hbm.at[s, pl.ds(c * CH, CH)], wt)
            ysrc = y_hbm

        def _z(i, _):
            for j in range(grp):
                ga[i, pl.ds(j * LANES, LANES)] = jnp.zeros((LANES,), jnp.float32)
            return 0
        lax.fori_loop(0, CE, _z, 0)
        base = s * rpt
        for q in range(rpt // CE):
            pltpu.sync_copy(ga, accum.at[pl.ds(base + q * CE, CE)])
        pltpu.async_copy(ysrc.at[rowt.at[0]], ga, gsa)
        pltpu.async_copy(ysrc.at[rowt.at[1]], gb, gsb)
        plsc.subcore_barrier()

        def scale(buf, k):
            def grp16(g, _):
                wv = wt[k, pl.ds(g * LANES, LANES)]
                ws = [_splat(wv, lane) for lane in range(LANES)]
                for lane in range(LANES):
                    e = g * LANES + lane
                    vals = [buf[e, pl.ds(j * LANES, LANES)] for j in range(grp)]
                    for j in range(grp):
                        buf[e, pl.ds(j * LANES, LANES)] = vals[j] * ws[lane]
                return 0
            lax.fori_loop(0, CE // LANES, grp16, 0)

        def pair(t, _):
            ka = 2 * t
            kb = 2 * t + 1
            pltpu.make_async_copy(ysrc.at[rowt.at[ka]], ga, gsa).wait()
            scale(ga, ka)
            pltpu.async_copy(ga, accum.at[colt.at[ka]], ssa, add=True)
            pltpu.make_async_copy(ysrc.at[rowt.at[kb]], gb, gsb).wait()
            scale(gb, kb)
            pltpu.async_copy(gb, accum.at[colt.at[kb]], ssb, add=True)
            ka2 = jnp.minimum(ka + 2, cht - 2)
            kb2 = jnp.minimum(kb + 2, cht - 1)
            pltpu.make_async_copy(ga, accum.at[colt.at[ka]], ssa).wait()
            pltpu.async_copy(ysrc.at[rowt.at[ka2]], ga, gsa)
            pltpu.make_async_copy(gb, accum.at[colt.at[kb]], ssb).wait()
            pltpu.async_copy(ysrc.at[rowt.at[kb2]], gb, gsb)
            return 0
        lax.fori_loop(0, cht // 2, pair, 0)
        pltpu.make_async_copy(ysrc.at[rowt.at[cht - 2]], ga, gsa).wait()
        pltpu.make_async_copy(ysrc.at[rowt.at[cht - 1]], gb, gsb).wait()
        plsc.subcore_barrier()
        pltpu.sync_copy(accum.at[pl.ds(base, rpt)],
                        out_hbm.at[c, pl.ds(base, rpt)])

    return pl.kernel(
        body,
        out_type=jax.ShapeDtypeStruct((NC, n, mh), jnp.float32),
        mesh=mesh,
        compiler_params=pltpu.CompilerParams(
            needs_layout_passes=False, use_tc_tiling_on_sc=False),
        scratch_types=[
            pltpu.VMEM((cht, CE), jnp.int32),
            pltpu.VMEM((cht, CE), jnp.int32),
            pltpu.VMEM((cht, CE), jnp.float32),
            pltpu.VMEM((CE, mh), jnp.float32),
            pltpu.VMEM((CE, mh), jnp.float32),
            pltpu.SemaphoreType.DMA,
            pltpu.SemaphoreType.DMA,
            pltpu.SemaphoreType.DMA,
            pltpu.SemaphoreType.DMA,
            pltpu.VMEM_SHARED((n, mh), jnp.float32),
        ],
    )


@functools.cache
def _make_rho(n):
    mesh = plsc.VectorSubcoreMesh(core_axis_name="c", subcore_axis_name="s")
    npad = NP
    spt = npad // NS

    def body(row_hbm, col_hbm, w_hbm, out_hbm,
             rowt, colt, wt, pb, vt, avt, zb, sb, pbt, sem, ssem, accum, parts):
        c = lax.axis_index("c")
        s = lax.axis_index("s")
        pltpu.sync_copy(row_hbm.at[s], rowt)
        pltpu.sync_copy(col_hbm.at[s], colt)
        pltpu.sync_copy(w_hbm.at[s], wt)

        v0 = jnp.full((LANES,), np.float32(1.0) / np.float32(np.sqrt(n)),
                      jnp.float32)
        zv = jnp.zeros((LANES,), jnp.float32)

        def _iv(i, _):
            vt[pl.ds(i * LANES, LANES)] = v0
            return 0
        lax.fori_loop(0, n // LANES, _iv, 0)

        def _ivz(i, _):
            vt[pl.ds(i * LANES, LANES)] = zv
            return 0
        lax.fori_loop(n // LANES, npad // LANES, _ivz, 0)
        for i in range(spt // LANES):
            zb[pl.ds(i * LANES, LANES)] = zv

        def it(t, s2_prev):
            pltpu.sync_copy(zb, accum.at[pl.ds(s * spt, spt)])
            plsc.subcore_barrier()

            def prow(r, _):
                for j in range(CE // LANES):
                    cv = colt[r, pl.ds(j * LANES, LANES)]
                    wv = wt[r, pl.ds(j * LANES, LANES)]
                    vv = plsc.load_gather(vt, [cv])
                    pb[r, pl.ds(j * LANES, LANES)] = vv * wv
                pltpu.async_copy(pb.at[r], accum.at[rowt.at[r]], ssem,
                                 add=True)
                return 0
            lax.fori_loop(0, 2 * CH, prow, 0)
            for r in range(2 * CH):
                pltpu.make_async_copy(pb.at[r], accum.at[rowt.at[r]],
                                      ssem).wait()
            plsc.subcore_barrier()

            pltpu.sync_copy(accum, avt)

            def sq(i, acc):
                x = avt[pl.ds(s * spt + i * LANES, LANES)]
                return acc + x * x
            part = lax.fori_loop(0, spt // LANES, sq,
                                 jnp.zeros((LANES,), jnp.float32))
            sb[...] = part
            pltpu.sync_copy(sb, parts.at[s])
            plsc.subcore_barrier()
            pltpu.sync_copy(parts, pbt)
            tot = jnp.zeros((LANES,), jnp.float32)
            for t2 in range(NS):
                tot = tot + pbt[t2]
            s2 = jnp.sum(tot)
            inv = _rsq(s2)

            def up(i, _):
                vt[pl.ds(i * LANES, LANES)] = avt[pl.ds(i * LANES, LANES)] * inv
                return 0
            lax.fori_loop(0, npad // LANES, up, 0)
            plsc.subcore_barrier()
            return s2
        s2f = lax.fori_loop(0, NITER, it, jnp.float32(0.0))
        rho = s2f * _rsq(s2f)

        @pl.when((c == 0) & (s == 0))
        def _():
            sb[...] = jnp.full((LANES,), np.float32(0.0)) + rho
            pltpu.sync_copy(sb, out_hbm)

    return pl.kernel(
        body,
        out_type=jax.ShapeDtypeStruct((LANES,), jnp.float32),
        mesh=mesh,
        compiler_params=pltpu.CompilerParams(
            needs_layout_passes=False, use_tc_tiling_on_sc=False),
        scratch_types=[
            pltpu.VMEM((2 * CH, CE), jnp.int32),
            pltpu.VMEM((2 * CH, CE), jnp.int32),
            pltpu.VMEM((2 * CH, CE), jnp.float32),
            pltpu.VMEM((2 * CH, CE), jnp.float32),
            pltpu.VMEM((npad,), jnp.float32),
            pltpu.VMEM((npad,), jnp.float32),
            pltpu.VMEM((spt,), jnp.float32),
            pltpu.VMEM((LANES,), jnp.float32),
            pltpu.VMEM((NS, LANES), jnp.float32),
            pltpu.SemaphoreType.DMA,
            pltpu.SemaphoreType.DMA,
            pltpu.VMEM_SHARED((npad,), jnp.float32),
            pltpu.VMEM_SHARED((NS, LANES), jnp.float32),
        ],
    )


def _store(o_ref, res, split, mh):
    if split:
        o_ref[0] = res[:, :mh]
        o_ref[1] = res[:, mh:]
    else:
        o_ref[...] = res


def _merge2(a_ref, b_ref, split):
    if split:
        return jnp.concatenate([a_ref[0] + b_ref[0], a_ref[1] + b_ref[1]],
                               axis=-1)
    return a_ref[0] + a_ref[1] + b_ref[0] + b_ref[1]


def _merge1(a_ref, split):
    if split:
        return jnp.concatenate([a_ref[0], a_ref[1]], axis=-1)
    return a_ref[0] + a_ref[1]


@functools.cache
def _mm(n, kin, kout, split):
    mh = kout // 2 if split else kout

    def body(z_ref, w_ref, o_ref):
        res = jnp.dot(z_ref[...], w_ref[...],
                      preferred_element_type=jnp.float32)
        _store(o_ref, res, split, mh)
    if split:
        out_spec = pl.BlockSpec((NC, BLK, mh), lambda i: (0, i, 0))
        out_shape = jax.ShapeDtypeStruct((NC, n, mh), jnp.float32)
    else:
        out_spec = pl.BlockSpec((BLK, kout), lambda i: (i, 0))
        out_shape = jax.ShapeDtypeStruct((n, kout), jnp.float32)
    return pl.pallas_call(
        body,
        grid=(n // BLK,),
        in_specs=[pl.BlockSpec((BLK, kin), lambda i: (i, 0)),
                  pl.BlockSpec((kin, kout), lambda i: (0, 0))],
        out_specs=out_spec,
        out_shape=out_shape,
    )


@functools.cache
def _relu_mm(n, m, with_p, split):
    mh = m // 2 if split else m

    def body(*refs):
        if with_p:
            p_ref, b_ref, w_ref, o_ref = refs
            x = _merge2(p_ref, b_ref, split)
        else:
            b_ref, w_ref, o_ref = refs
            x = _merge1(b_ref, split)
        res = jnp.dot(jnp.maximum(x, 0.0), w_ref[...],
                      preferred_element_type=jnp.float32)
        _store(o_ref, res, split, mh)
    pspec = pl.BlockSpec((NC, BLK, mh), lambda i: (0, i, 0))
    in_specs = ([pspec, pspec] if with_p else [pspec]) + [
        pl.BlockSpec((m, m), lambda i: (0, 0))]
    if split:
        out_spec = pl.BlockSpec((NC, BLK, mh), lambda i: (0, i, 0))
        out_shape = jax.ShapeDtypeStruct((NC, n, mh), jnp.float32)
    else:
        out_spec = pl.BlockSpec((BLK, m), lambda i: (i, 0))
        out_shape = jax.ShapeDtypeStruct((n, m), jnp.float32)
    return pl.pallas_call(
        body,
        grid=(n // BLK,),
        in_specs=in_specs,
        out_specs=out_spec,
        out_shape=out_shape,
    )


@functools.cache
def _comb(n, m, kin, act, split):
    mh = m // 2 if split else m

    def body(p_ref, b_ref, z_ref, w_ref, bias_ref, o_ref):
        x = jnp.maximum(_merge2(p_ref, b_ref, split), 0.0)
        y = x + jnp.dot(z_ref[...], w_ref[...],
                        preferred_element_type=jnp.float32) + bias_ref[...]
        o_ref[...] = jnp.where(y > 0, y, jnp.exp(y) - 1.0) if act else y
    pspec = pl.BlockSpec((NC, BLK, mh), lambda i: (0, i, 0))
    return pl.pallas_call(
        body,
        grid=(n // BLK,),
        in_specs=[pspec, pspec,
                  pl.BlockSpec((BLK, kin), lambda i: (i, 0)),
                  pl.BlockSpec((kin, m), lambda i: (0, 0)),
                  pl.BlockSpec((1, m), lambda i: (0, 0))],
        out_specs=pl.BlockSpec((BLK, m), lambda i: (i, 0)),
        out_shape=jax.ShapeDtypeStruct((n, m), jnp.float32),
    )


def _proj(W, v):
    a = jnp.abs(W)
    asort = jnp.sort(a, axis=1)[:, ::-1]
    cssv = jnp.cumsum(asort, axis=1) - v
    ind = jnp.arange(1, W.shape[1] + 1, dtype=W.dtype)
    cond = (asort - cssv / ind) > 0
    rho = jnp.maximum(jnp.sum(cond, axis=1), 1)
    theta = cssv[jnp.arange(W.shape[0]), rho - 1] / rho.astype(W.dtype)
    proj = jnp.sign(W) * jnp.maximum(a - theta[:, None], 0.0)
    return jnp.where((jnp.sum(a, axis=1) > v)[:, None], proj, W)


def _pack_edges(x, pad):
    x = jnp.pad(x.reshape(NW, -1), ((0, 0), (0, pad))).reshape(NW, CH, CE)
    return x.reshape(2, NS, CH, CE).transpose(1, 0, 2, 3).reshape(NS, 2 * CH, CE)


def kernel(features, edge_index, edge_weight, W1, O1, W2, O2, W3, O3, W4, O4,
           W5, O5, V0w, V0b, V1w, V1b, V2w, V2b, V3w, V3b, Vw, Vb):
    n = features.shape[1]
    e = edge_weight.shape[0]
    pad = CH * CE - e // NW
    rowg = _pack_edges(edge_index[0], pad)
    colg = _pack_edges(edge_index[1], pad)
    wg = _pack_edges(edge_weight, pad)

    a_rho = _make_rho(n)(rowg, colg, wg)[0]
    radius = KAPPA / a_rho

    z = jnp.pad(features.T, ((0, NP - n), (0, 0)))
    layers = [(W1, O1, V0w, V0b), (W2, O2, V1w, V1b), (W3, O3, V2w, V2b),
              (W4, O4, V3w, V3b), (W5, O5, Vw, Vb)]
    for li, (W, O, Vw_, Vb_) in enumerate(layers):
        m, p = O.shape
        split = m >= 32
        Wp = _proj(W, radius)
        spmm = _make_spmm(NP, m, split)
        s2 = _mm(NP, p, m, split)(z, O.T)
        b2 = spmm(s2, rowg, colg, wg)
        h = _relu_mm(NP, m, False, split)(b2, Wp.T)
        for _ in range(8):
            p2 = spmm(h, rowg, colg, wg)
            h = _relu_mm(NP, m, True, split)(p2, b2, Wp.T)
        p2 = spmm(h, rowg, colg, wg)
        z = _comb(NP, m, p, li < 4, split)(p2, b2, z, Vw_.T, Vb_.reshape(1, m))
    return z[:n]


# speedup vs baseline: 1.2372x; 1.0740x over previous
import functools

import numpy as np
import jax
import jax.numpy as jnp
from jax import lax
from jax.experimental import pallas as pl
from jax.experimental.pallas import tpu as pltpu
from jax.experimental.pallas import tpu_sc as plsc

KAPPA = 0.9
NC, NS, LANES = 2, 16, 16
NW = NC * NS
CE = 128
CH = 80
NITER = 50
NP = 10240
BLK = 512

_GDN = lax.GatherDimensionNumbers(
    offset_dims=(), collapsed_slice_dims=(0,), start_index_map=(0,))


def _splat(vec, lane):
    idx = jnp.full((LANES, 1), lane, jnp.int32)
    return lax.gather(vec, idx, _GDN, slice_sizes=(1,),
                      mode=lax.GatherScatterMode.PROMISE_IN_BOUNDS)


def _rsq(x):
    xs = jnp.maximum(x, np.float32(1e-30))
    i = lax.bitcast_convert_type(xs, jnp.int32)
    i = np.int32(0x5F3759DF) - lax.shift_right_logical(i, 1)
    y = lax.bitcast_convert_type(i, jnp.float32)
    for _ in range(4):
        y = y * (np.float32(1.5) - np.float32(0.5) * xs * y * y)
    return y


@functools.cache
def _make_spmm(n, m, split):
    mesh = plsc.VectorSubcoreMesh(core_axis_name="c", subcore_axis_name="s")
    if split:
        mh = 16 if m == 32 else 32
        nf = m // mh
        ph = nf // 2
        cht = 2 * CH
    else:
        mh = m
        nf = 2
        ph = 1
        cht = CH
    rpt = n // NS
    grp = mh // LANES

    def body(y_hbm, row_hbm, col_hbm, w_hbm, out_hbm,
             rowt, colt, wt, g0, g1, g2, g3, gs0, gs1, gs2, gs3,
             ss0, ss1, ss2, ss3, accum):
        gbufs = (g0, g1, g2, g3)
        gsems = (gs0, gs1, gs2, gs3)
        ssems = (ss0, ss1, ss2, ss3)
        c = lax.axis_index("c")
        s = lax.axis_index("s")
        if split:
            pltpu.sync_copy(row_hbm.at[s], rowt)
            pltpu.sync_copy(col_hbm.at[s], colt)
            pltpu.sync_copy(w_hbm.at[s], wt)
        else:
            pltpu.sync_copy(row_hbm.at[s, pl.ds(c * CH, CH)], rowt)
            pltpu.sync_copy(col_hbm.at[s, pl.ds(c * CH, CH)], colt)
            pltpu.sync_copy(w_hbm.at[s, pl.ds(c * CH, CH)], wt)
        base = s * rpt

        def scale(buf, k):
            def grp16(g, _):
                wv = wt[k, pl.ds(g * LANES, LANES)]
                ws = [_splat(wv, lane) for lane in range(LANES)]
                for lane in range(LANES):
                    e = g * LANES + lane
                    vals = [buf[e, pl.ds(j * LANES, LANES)] for j in range(grp)]
                    for j in range(grp):
                        buf[e, pl.ds(j * LANES, LANES)] = vals[j] * ws[lane]
                return 0
            lax.fori_loop(0, CE // LANES, grp16, 0)

        for p in range(ph):
            if split:
                stk = 2 * p + c
                ysrc = y_hbm.at[stk]
            else:
                stk = c
                ysrc = y_hbm
            def _z(i, _):
                for j in range(grp):
                    g0[i, pl.ds(j * LANES, LANES)] = jnp.zeros((LANES,),
                                                               jnp.float32)
                return 0
            lax.fori_loop(0, CE, _z, 0)
            for q in range(rpt // CE):
                pltpu.sync_copy(g0, accum.at[pl.ds(base + q * CE, CE)])
            for i in range(4):
                pltpu.async_copy(ysrc.at[rowt.at[i]], gbufs[i], gsems[i])
            plsc.subcore_barrier()

            def quad(t, _):
                k0 = 4 * t
                for i in range(4):
                    k = k0 + i
                    pltpu.make_async_copy(ysrc.at[rowt.at[k]], gbufs[i],
                                          gsems[i]).wait()
                    scale(gbufs[i], k)
                    pltpu.async_copy(gbufs[i], accum.at[colt.at[k]], ssems[i],
                                     add=True)
                for i in range(4):
                    k = k0 + i
                    pltpu.make_async_copy(gbufs[i], accum.at[colt.at[k]],
                                          ssems[i]).wait()
                    kn = jnp.minimum(k + 4, cht - 4 + i)
                    pltpu.async_copy(ysrc.at[rowt.at[kn]], gbufs[i], gsems[i])
                return 0
            lax.fori_loop(0, cht // 4, quad, 0)
            for i in range(4):
                pltpu.make_async_copy(ysrc.at[rowt.at[cht - 4 + i]], gbufs[i],
                                      gsems[i]).wait()
            plsc.subcore_barrier()
            pltpu.sync_copy(accum.at[pl.ds(base, rpt)],
                            out_hbm.at[stk, pl.ds(base, rpt)])

    return pl.kernel(
        body,
        out_type=jax.ShapeDtypeStruct((nf, n, mh), jnp.float32),
        mesh=mesh,
        compiler_params=pltpu.CompilerParams(
            needs_layout_passes=False, use_tc_tiling_on_sc=False),
        scratch_types=[
            pltpu.VMEM((cht, CE), jnp.int32),
            pltpu.VMEM((cht, CE), jnp.int32),
            pltpu.VMEM((cht, CE), jnp.float32),
            pltpu.VMEM((CE, mh), jnp.float32),
            pltpu.VMEM((CE, mh), jnp.float32),
            pltpu.VMEM((CE, mh), jnp.float32),
            pltpu.VMEM((CE, mh), jnp.float32),
            pltpu.SemaphoreType.DMA,
            pltpu.SemaphoreType.DMA,
            pltpu.SemaphoreType.DMA,
            pltpu.SemaphoreType.DMA,
            pltpu.SemaphoreType.DMA,
            pltpu.SemaphoreType.DMA,
            pltpu.SemaphoreType.DMA,
            pltpu.SemaphoreType.DMA,
            pltpu.VMEM_SHARED((n, mh), jnp.float32),
        ],
    )


@functools.cache
def _make_rho(n):
    mesh = plsc.VectorSubcoreMesh(core_axis_name="c", subcore_axis_name="s")
    npad = NP
    spt = npad // NS

    def body(row_hbm, col_hbm, w_hbm, out_hbm,
             rowt, colt, wt, pb, vt, avt, zb, sb, pbt, sem, ssem, accum, parts):
        c = lax.axis_index("c")
        s = lax.axis_index("s")
        pltpu.sync_copy(row_hbm.at[s], rowt)
        pltpu.sync_copy(col_hbm.at[s], colt)
        pltpu.sync_copy(w_hbm.at[s], wt)

        v0 = jnp.full((LANES,), np.float32(1.0) / np.float32(np.sqrt(n)),
                      jnp.float32)
        zv = jnp.zeros((LANES,), jnp.float32)

        def _iv(i, _):
            vt[pl.ds(i * LANES, LANES)] = v0
            return 0
        lax.fori_loop(0, n // LANES, _iv, 0)

        def _ivz(i, _):
            vt[pl.ds(i * LANES, LANES)] = zv
            return 0
        lax.fori_loop(n // LANES, npad // LANES, _ivz, 0)
        for i in range(spt // LANES):
            zb[pl.ds(i * LANES, LANES)] = zv

        def it(t, s2_prev):
            pltpu.sync_copy(zb, accum.at[pl.ds(s * spt, spt)])
            plsc.subcore_barrier()

            def prow(r, _):
                for j in range(CE // LANES):
                    cv = colt[r, pl.ds(j * LANES, LANES)]
                    wv = wt[r, pl.ds(j * LANES, LANES)]
                    vv = plsc.load_gather(vt, [cv])
                    pb[r, pl.ds(j * LANES, LANES)] = vv * wv
                pltpu.async_copy(pb.at[r], accum.at[rowt.at[r]], ssem,
                                 add=True)
                return 0
            lax.fori_loop(0, 2 * CH, prow, 0)
            for r in range(2 * CH):
                pltpu.make_async_copy(pb.at[r], accum.at[rowt.at[r]],
                                      ssem).wait()
            plsc.subcore_barrier()

            pltpu.sync_copy(accum, avt)

            def sq(i, acc):
                x = avt[pl.ds(s * spt + i * LANES, LANES)]
                return acc + x * x
            part = lax.fori_loop(0, spt // LANES, sq,
                                 jnp.zeros((LANES,), jnp.float32))
            sb[...] = part
            pltpu.sync_copy(sb, parts.at[s])
            plsc.subcore_barrier()
            pltpu.sync_copy(parts, pbt)
            tot = jnp.zeros((LANES,), jnp.float32)
            for t2 in range(NS):
                tot = tot + pbt[t2]
            s2 = jnp.sum(tot)
            inv = _rsq(s2)

            def up(i, _):
                vt[pl.ds(i * LANES, LANES)] = avt[pl.ds(i * LANES, LANES)] * inv
                return 0
            lax.fori_loop(0, npad // LANES, up, 0)
            plsc.subcore_barrier()
            return s2
        s2f = lax.fori_loop(0, NITER, it, jnp.float32(0.0))
        rho = s2f * _rsq(s2f)

        @pl.when((c == 0) & (s == 0))
        def _():
            sb[...] = jnp.full((LANES,), np.float32(0.0)) + rho
            pltpu.sync_copy(sb, out_hbm)

    return pl.kernel(
        body,
        out_type=jax.ShapeDtypeStruct((LANES,), jnp.float32),
        mesh=mesh,
        compiler_params=pltpu.CompilerParams(
            needs_layout_passes=False, use_tc_tiling_on_sc=False),
        scratch_types=[
            pltpu.VMEM((2 * CH, CE), jnp.int32),
            pltpu.VMEM((2 * CH, CE), jnp.int32),
            pltpu.VMEM((2 * CH, CE), jnp.float32),
            pltpu.VMEM((2 * CH, CE), jnp.float32),
            pltpu.VMEM((npad,), jnp.float32),
            pltpu.VMEM((npad,), jnp.float32),
            pltpu.VMEM((spt,), jnp.float32),
            pltpu.VMEM((LANES,), jnp.float32),
            pltpu.VMEM((NS, LANES), jnp.float32),
            pltpu.SemaphoreType.DMA,
            pltpu.SemaphoreType.DMA,
            pltpu.VMEM_SHARED((npad,), jnp.float32),
            pltpu.VMEM_SHARED((NS, LANES), jnp.float32),
        ],
    )


def _nf_mh(m, split):
    if split:
        mh = 16 if m == 32 else 32
        return m // mh, mh
    return 2, m


def _store(o_ref, res, split, nf, mh):
    if split:
        for q in range(nf):
            o_ref[q] = res[:, q * mh:(q + 1) * mh]
    else:
        o_ref[...] = res


def _merge2(a_ref, b_ref, split, nf):
    if split:
        return jnp.concatenate([a_ref[q] + b_ref[q] for q in range(nf)],
                               axis=-1)
    return a_ref[0] + a_ref[1] + b_ref[0] + b_ref[1]


def _merge1(a_ref, split, nf):
    if split:
        return jnp.concatenate([a_ref[q] for q in range(nf)], axis=-1)
    return a_ref[0] + a_ref[1]


@functools.cache
def _mm(n, kin, kout, split):
    nf, mh = _nf_mh(kout, split)

    def body(z_ref, w_ref, o_ref):
        res = jnp.dot(z_ref[...], w_ref[...],
                      preferred_element_type=jnp.float32)
        _store(o_ref, res, split, nf, mh)
    if split:
        out_spec = pl.BlockSpec((nf, BLK, mh), lambda i: (0, i, 0))
        out_shape = jax.ShapeDtypeStruct((nf, n, mh), jnp.float32)
    else:
        out_spec = pl.BlockSpec((BLK, kout), lambda i: (i, 0))
        out_shape = jax.ShapeDtypeStruct((n, kout), jnp.float32)
    return pl.pallas_call(
        body,
        grid=(n // BLK,),
        in_specs=[pl.BlockSpec((BLK, kin), lambda i: (i, 0)),
                  pl.BlockSpec((kin, kout), lambda i: (0, 0))],
        out_specs=out_spec,
        out_shape=out_shape,
    )


@functools.cache
def _relu_mm(n, m, with_p, split):
    nf, mh = _nf_mh(m, split)

    def body(*refs):
        if with_p:
            p_ref, b_ref, w_ref, o_ref = refs
            x = _merge2(p_ref, b_ref, split, nf)
        else:
            b_ref, w_ref, o_ref = refs
            x = _merge1(b_ref, split, nf)
        res = jnp.dot(jnp.maximum(x, 0.0), w_ref[...],
                      preferred_element_type=jnp.float32)
        _store(o_ref, res, split, nf, mh)
    pspec = pl.BlockSpec((nf, BLK, mh), lambda i: (0, i, 0))
    in_specs = ([pspec, pspec] if with_p else [pspec]) + [
        pl.BlockSpec((m, m), lambda i: (0, 0))]
    if split:
        out_spec = pl.BlockSpec((nf, BLK, mh), lambda i: (0, i, 0))
        out_shape = jax.ShapeDtypeStruct((nf, n, mh), jnp.float32)
    else:
        out_spec = pl.BlockSpec((BLK, m), lambda i: (i, 0))
        out_shape = jax.ShapeDtypeStruct((n, m), jnp.float32)
    return pl.pallas_call(
        body,
        grid=(n // BLK,),
        in_specs=in_specs,
        out_specs=out_spec,
        out_shape=out_shape,
    )


@functools.cache
def _comb(n, m, kin, act, split):
    nf, mh = _nf_mh(m, split)

    def body(p_ref, b_ref, z_ref, w_ref, bias_ref, o_ref):
        x = jnp.maximum(_merge2(p_ref, b_ref, split, nf), 0.0)
        y = x + jnp.dot(z_ref[...], w_ref[...],
                        preferred_element_type=jnp.float32) + bias_ref[...]
        o_ref[...] = jnp.where(y > 0, y, jnp.exp(y) - 1.0) if act else y
    pspec = pl.BlockSpec((nf, BLK, mh), lambda i: (0, i, 0))
    return pl.pallas_call(
        body,
        grid=(n // BLK,),
        in_specs=[pspec, pspec,
                  pl.BlockSpec((BLK, kin), lambda i: (i, 0)),
                  pl.BlockSpec((kin, m), lambda i: (0, 0)),
                  pl.BlockSpec((1, m), lambda i: (0, 0))],
        out_specs=pl.BlockSpec((BLK, m), lambda i: (i, 0)),
        out_shape=jax.ShapeDtypeStruct((n, m), jnp.float32),
    )


def _proj(W, v):
    a = jnp.abs(W)
    asort = jnp.sort(a, axis=1)[:, ::-1]
    cssv = jnp.cumsum(asort, axis=1) - v
    ind = jnp.arange(1, W.shape[1] + 1, dtype=W.dtype)
    cond = (asort - cssv / ind) > 0
    rho = jnp.maximum(jnp.sum(cond, axis=1), 1)
    theta = cssv[jnp.arange(W.shape[0]), rho - 1] / rho.astype(W.dtype)
    proj = jnp.sign(W) * jnp.maximum(a - theta[:, None], 0.0)
    return jnp.where((jnp.sum(a, axis=1) > v)[:, None], proj, W)


def _pack_edges(x, pad):
    x = jnp.pad(x.reshape(NW, -1), ((0, 0), (0, pad))).reshape(NW, CH, CE)
    return x.reshape(2, NS, CH, CE).transpose(1, 0, 2, 3).reshape(NS, 2 * CH, CE)


def kernel(features, edge_index, edge_weight, W1, O1, W2, O2, W3, O3, W4, O4,
           W5, O5, V0w, V0b, V1w, V1b, V2w, V2b, V3w, V3b, Vw, Vb):
    n = features.shape[1]
    e = edge_weight.shape[0]
    pad = CH * CE - e // NW
    rowg = _pack_edges(edge_index[0], pad)
    colg = _pack_edges(edge_index[1], pad)
    wg = _pack_edges(edge_weight, pad)

    a_rho = _make_rho(n)(rowg, colg, wg)[0]
    radius = KAPPA / a_rho

    z = jnp.pad(features.T, ((0, NP - n), (0, 0)))
    layers = [(W1, O1, V0w, V0b), (W2, O2, V1w, V1b), (W3, O3, V2w, V2b),
              (W4, O4, V3w, V3b), (W5, O5, Vw, Vb)]
    for li, (W, O, Vw_, Vb_) in enumerate(layers):
        m, p = O.shape
        split = m >= 32
        Wp = _proj(W, radius)
        spmm = _make_spmm(NP, m, split)
        s2 = _mm(NP, p, m, split)(z, O.T)
        b2 = spmm(s2, rowg, colg, wg)
        h = _relu_mm(NP, m, False, split)(b2, Wp.T)
        for _ in range(8):
            p2 = spmm(h, rowg, colg, wg)
            h = _relu_mm(NP, m, True, split)(p2, b2, Wp.T)
        p2 = spmm(h, rowg, colg, wg)
        z = _comb(NP, m, p, li < 4, split)(p2, b2, z, Vw_.T, Vb_.reshape(1, m))
    return z[:n]
